# baseline (speedup 1.0000x reference)
import functools

import jax
import jax.numpy as jnp
from jax import lax
from jax.experimental import pallas as pl
from jax.experimental.pallas import tpu as pltpu
from jax.experimental.pallas import tpu_sc as plsc

_DIM = 32
_G = 128
_NC = 2
_NS = 16
_NW = _NC * _NS
_CB = 2048
_PB = 8192
_CH = 512


def _prep_table(table_t, v):
    q4 = _CB // 4
    sb = _PB // _CB
    grid = (v + _PB - 1) // _PB

    def body(in_ref, out_ref):
        x = in_ref[...]
        rows = [
            jnp.concatenate(
                [x[:, s * _CB + q * q4:s * _CB + (q + 1) * q4] for s in range(sb)],
                axis=1,
            )
            for q in range(4)
        ]
        y = jnp.concatenate(rows, axis=0)
        out_ref[...] = jnp.transpose(y)

    return pl.pallas_call(
        body,
        grid=(grid,),
        in_specs=[pl.BlockSpec((_DIM, _PB), lambda c: (0, c))],
        out_specs=pl.BlockSpec((_PB // 4, 128), lambda c: (c, 0)),
        out_shape=jax.ShapeDtypeStruct((grid * _PB // 4, 128), jnp.float32),
    )(table_t)


@functools.partial(jax.jit, static_argnums=(2,))
def _gather_lines(idx2d, table, n_rows):
    chunks = n_rows * _G // _CH
    pairs_per_w = chunks // _NW // 2
    n_lines = n_rows * _G // 4
    mesh = plsc.VectorSubcoreMesh(core_axis_name="c", subcore_axis_name="s")

    @functools.partial(
        pl.kernel,
        mesh=mesh,
        out_type=jax.ShapeDtypeStruct((n_lines, 128), jnp.float32),
        scratch_types=[
            pltpu.VMEM((8, _G), jnp.int32),
            pltpu.VMEM((_CH, _DIM), jnp.float32),
            pltpu.VMEM((_CH, _DIM), jnp.float32),
            pltpu.VMEM((128, 129), jnp.float32),
            pltpu.VMEM((128, 129), jnp.float32),
            pltpu.SemaphoreType.DMA,
            pltpu.SemaphoreType.DMA,
            pltpu.SemaphoreType.DMA,
            pltpu.SemaphoreType.DMA,
        ],
        compiler_params=pltpu.CompilerParams(use_tc_tiling_on_sc=False, needs_layout_passes=False),
    )
    def gather(idx_hbm, table_hbm, out_hbm, idx_v, buf_a, buf_b,
               line_a, line_b, sem_ga, sem_gb, sem_sa, sem_sb):
        wid = lax.axis_index("s") * _NC + lax.axis_index("c")
        p0 = wid * pairs_per_w
        i16 = jax.lax.iota(jnp.int32, 16)
        rv = [(((i16 + 16 * h) // 8) * 32 + (i16 + 16 * h) % 8) for h in (0, 1)]

        def transpose_chunk(buf, line):
            def grp(g, carry):
                bt8 = (g // 8) * 8
                col0 = (g % 8) * 16
                row_idx = [rv[0] + bt8, rv[1] + bt8]
                for k in range(16):
                    m = g * 16 + k
                    col = jnp.full((16,), col0 + k, jnp.int32)
                    for h in (0, 1):
                        plsc.store_scatter(
                            line, [row_idx[h], col], buf[m, pl.ds(16 * h, 16)])
                return carry

            lax.fori_loop(0, _CH // 16, grp, 0)

        def line_dmas(line, c, sem):
            l_idx = c // 32
            bq = (c % 32) * 32
            return [
                pltpu.async_copy(
                    line.at[pl.ds(dt * 32, 32), pl.ds(0, 128)],
                    out_hbm.at[pl.ds((l_idx * 4 + dt) * 1024 + bq, 32), :],
                    sem,
                )
                for dt in range(4)
            ]

        def drain_lines(line, c, sem):
            l_idx = c // 32
            bq = (c % 32) * 32
            for dt in range(4):
                pltpu.make_async_copy(
                    line.at[pl.ds(dt * 32, 32), pl.ds(0, 128)],
                    out_hbm.at[pl.ds((l_idx * 4 + dt) * 1024 + bq, 32), :],
                    sem,
                ).wait()

        def body(i, carry):
            pair = p0 + i
            ca_id = pair * 2
            cb_id = ca_id + 1

            @pl.when(i > 0)
            def _():
                drain_lines(line_a, ca_id, sem_sa)
                drain_lines(line_b, cb_id, sem_sb)

            pltpu.sync_copy(idx_hbm.at[pl.ds(ca_id * 4, 8)], idx_v)
            ca = [
                pltpu.async_copy(
                    table_hbm.at[idx_v.at[j]],
                    buf_a.at[pl.ds(j * _G, _G)], sem_ga)
                for j in range(4)
            ]
            cb = [
                pltpu.async_copy(
                    table_hbm.at[idx_v.at[4 + j]],
                    buf_b.at[pl.ds(j * _G, _G)], sem_gb)
                for j in range(4)
            ]
            for cp in ca:
                cp.wait()
            transpose_chunk(buf_a, line_a)
            line_dmas(line_a, ca_id, sem_sa)
            for cp in cb:
                cp.wait()
            transpose_chunk(buf_b, line_b)
            line_dmas(line_b, cb_id, sem_sb)
            return carry

        lax.fori_loop(0, pairs_per_w, body, 0)
        last = p0 + pairs_per_w - 1
        drain_lines(line_a, last * 2, sem_sa)
        drain_lines(line_b, last * 2 + 1, sem_sb)

    return gather(idx2d, table)


def kernel(indices, table):
    b, l = indices.shape
    n = b * l
    v = table.shape[0]
    vp = ((v + _PB - 1) // _PB) * _PB
    table_lin = _prep_table(jnp.transpose(table), v).reshape(vp, _DIM)
    idx = jnp.transpose(indices).astype(jnp.int32)
    jp = (idx & ~2047) + ((idx & 511) << 2) + ((idx & 2047) >> 9)
    idx2d = jp.reshape(n // _G, _G)
    g128 = _gather_lines(idx2d, table_lin, n // _G)
    y = g128.reshape(l, 4, 128, 8, 128)
    y = jnp.transpose(y, (0, 1, 3, 2, 4)).reshape(l, _DIM, b)
    return jnp.transpose(y, (2, 0, 1))


# speedup vs baseline: 1.6194x; 1.6194x over previous
import functools

import jax
import jax.numpy as jnp
from jax import lax
from jax.experimental import pallas as pl
from jax.experimental.pallas import tpu as pltpu
from jax.experimental.pallas import tpu_sc as plsc

_DIM = 32
_G = 128
_CHUNK = 2048
_NC = 2
_NS = 16
_NW = _NC * _NS
_CB = 2048
_PB = 8192
_BC = 16384
_P = 10


def _prep_table(table_t, v):
    q4 = _CB // 4
    sb = _PB // _CB
    grid = (v + _PB - 1) // _PB

    def body(in_ref, out_ref):
        x = in_ref[...]
        rows = [
            jnp.concatenate(
                [x[:, s * _CB + q * q4:s * _CB + (q + 1) * q4] for s in range(sb)],
                axis=1,
            )
            for q in range(4)
        ]
        y = jnp.concatenate(rows, axis=0)
        out_ref[...] = jnp.transpose(y)

    return pl.pallas_call(
        body,
        grid=(grid,),
        in_specs=[pl.BlockSpec((_DIM, _PB), lambda c: (0, c))],
        out_specs=pl.BlockSpec((_PB // 4, 128), lambda c: (c, 0)),
        out_shape=jax.ShapeDtypeStruct((grid * _PB // 4, 128), jnp.float32),
    )(table_t)


def _transpose_piece(g128_p, acc, piece, l_pp, l, b):
    sb = _BC // _CHUNK

    def body(*refs):
        in_ref, out_ref = refs[0], refs[-1]
        xt = jnp.transpose(in_ref[...])
        for c in range(sb):
            for q in range(4):
                out_ref[0, :, c * _CHUNK + 512 * q:c * _CHUNK + 512 * (q + 1)] = (
                    xt[32 * q:32 * (q + 1), c * 512:(c + 1) * 512])

    in_specs = [pl.BlockSpec((_BC // 4, 128), lambda i: (i, 0))]
    operands = [g128_p]
    kwargs = {}
    if acc is not None:
        in_specs.append(pl.BlockSpec(memory_space=pl.ANY))
        operands.append(acc)
        kwargs["input_output_aliases"] = {1: 0}

    return pl.pallas_call(
        body,
        grid=(l_pp,),
        in_specs=in_specs,
        out_specs=pl.BlockSpec((1, _DIM, _BC), lambda i, piece=piece: (piece * l_pp + i, 0, 0)),
        out_shape=jax.ShapeDtypeStruct((l, _DIM, b), jnp.float32),
        **kwargs,
    )(*operands)


@functools.partial(jax.jit, static_argnums=(2, 3, 4))
def _gather_rows(idx2d, table, n_rows, piece, n_pieces):
    rows_per_chunk = _CHUNK // _G
    chunks = n_rows // rows_per_chunk
    chunks_pp = chunks // n_pieces
    chunks_per_w = chunks_pp // _NW
    n_lines = chunks_pp * 512
    mesh = plsc.VectorSubcoreMesh(core_axis_name="c", subcore_axis_name="s")

    @functools.partial(
        pl.kernel,
        mesh=mesh,
        out_type=jax.ShapeDtypeStruct((n_lines, 128), jnp.float32),
        scratch_types=[
            pltpu.VMEM((rows_per_chunk, _G), jnp.int32),
            pltpu.VMEM((_CHUNK, _DIM), jnp.float32),
            pltpu.SemaphoreType.DMA,
        ],
        compiler_params=pltpu.CompilerParams(use_tc_tiling_on_sc=False),
    )
    def gather(idx_hbm, table_hbm, out_hbm, idx_v, rows_v, sem):
        wid = lax.axis_index("s") * _NC + lax.axis_index("c")
        c0 = piece * chunks_pp + wid * chunks_per_w

        def body(i, carry):
            c = c0 + i
            co = (wid * chunks_per_w + i) * 512
            pltpu.sync_copy(idx_hbm.at[pl.ds(c * rows_per_chunk, rows_per_chunk)], idx_v)
            copies = [
                pltpu.async_copy(
                    table_hbm.at[idx_v.at[j]],
                    rows_v.at[pl.ds(j * _G, _G)],
                    sem,
                )
                for j in range(rows_per_chunk)
            ]
            for cp in copies:
                cp.wait()
            for q in range(4):
                pltpu.sync_copy(
                    rows_v.at[pl.ds(512 * q, 512)],
                    out_hbm.at[pl.ds(co, 512), pl.ds(32 * q, 32)],
                )
            return carry

        lax.fori_loop(0, chunks_per_w, body, 0)

    return gather(idx2d, table)


def kernel(indices, table):
    b, l = indices.shape
    n = b * l
    v = table.shape[0]
    vp = ((v + _PB - 1) // _PB) * _PB
    table_lin = _prep_table(jnp.transpose(table), v).reshape(vp, _DIM)
    idx = jnp.transpose(indices).astype(jnp.int32)
    jp = (idx & ~2047) + ((idx & 511) << 2) + ((idx & 2047) >> 9)
    idx2d = jp.reshape(n // _G, _G)
    l_pp = l // _P
    acc = None
    for p in range(_P):
        g128_p = _gather_rows(idx2d, table_lin, n // _G, p, _P)
        acc = _transpose_piece(g128_p, acc, p, l_pp, l, b)
    return jnp.transpose(acc, (2, 0, 1))


# speedup vs baseline: 1.6496x; 1.0186x over previous
import functools

import jax
import jax.numpy as jnp
from jax import lax
from jax.experimental import pallas as pl
from jax.experimental.pallas import tpu as pltpu
from jax.experimental.pallas import tpu_sc as plsc

_DIM = 32
_G = 128
_CHUNK = 2048
_NC = 2
_NS = 16
_NW = _NC * _NS
_CB = 2048
_PB = 8192
_BC = 16384
_P = 5


def _prep_table(table_t, v):
    q4 = _CB // 4
    sb = _PB // _CB
    grid = (v + _PB - 1) // _PB

    def body(in_ref, out_ref):
        x = in_ref[...]
        rows = [
            jnp.concatenate(
                [x[:, s * _CB + q * q4:s * _CB + (q + 1) * q4] for s in range(sb)],
                axis=1,
            )
            for q in range(4)
        ]
        y = jnp.concatenate(rows, axis=0)
        out_ref[...] = jnp.transpose(y)

    return pl.pallas_call(
        body,
        grid=(grid,),
        in_specs=[pl.BlockSpec((_DIM, _PB), lambda c: (0, c))],
        out_specs=pl.BlockSpec((_PB // 4, 128), lambda c: (c, 0)),
        out_shape=jax.ShapeDtypeStruct((grid * _PB // 4, 128), jnp.float32),
    )(table_t)


def _transpose_piece(g128_p, acc, piece, l_pp, l, b):
    sb = _BC // _CHUNK

    def body(*refs):
        in_ref, out_ref = refs[0], refs[-1]
        xt = jnp.transpose(in_ref[...])
        for c in range(sb):
            for q in range(4):
                out_ref[0, :, c * _CHUNK + 512 * q:c * _CHUNK + 512 * (q + 1)] = (
                    xt[32 * q:32 * (q + 1), c * 512:(c + 1) * 512])

    in_specs = [pl.BlockSpec((_BC // 4, 128), lambda i: (i, 0))]
    operands = [g128_p]
    kwargs = {}
    if acc is not None:
        in_specs.append(pl.BlockSpec(memory_space=pl.ANY))
        operands.append(acc)
        kwargs["input_output_aliases"] = {1: 0}

    return pl.pallas_call(
        body,
        grid=(l_pp,),
        in_specs=in_specs,
        out_specs=pl.BlockSpec((1, _DIM, _BC), lambda i, piece=piece: (piece * l_pp + i, 0, 0)),
        out_shape=jax.ShapeDtypeStruct((l, _DIM, b), jnp.float32),
        **kwargs,
    )(*operands)


@functools.partial(jax.jit, static_argnums=(2, 3, 4))
def _gather_rows(idx2d, table, n_rows, piece, n_pieces):
    rows_per_chunk = _CHUNK // _G
    chunks = n_rows // rows_per_chunk
    chunks_pp = chunks // n_pieces
    chunks_per_w = chunks_pp // _NW
    n_lines = chunks_pp * 512
    mesh = plsc.VectorSubcoreMesh(core_axis_name="c", subcore_axis_name="s")

    @functools.partial(
        pl.kernel,
        mesh=mesh,
        out_type=jax.ShapeDtypeStruct((n_lines, 128), jnp.float32),
        scratch_types=[
            pltpu.VMEM((rows_per_chunk, _G), jnp.int32),
            pltpu.VMEM((_CHUNK, _DIM), jnp.float32),
            pltpu.SemaphoreType.DMA,
        ],
        compiler_params=pltpu.CompilerParams(use_tc_tiling_on_sc=False),
    )
    def gather(idx_hbm, table_hbm, out_hbm, idx_v, rows_v, sem):
        wid = lax.axis_index("s") * _NC + lax.axis_index("c")
        c0 = piece * chunks_pp + wid * chunks_per_w

        def body(i, carry):
            c = c0 + i
            co = (wid * chunks_per_w + i) * 512
            pltpu.sync_copy(idx_hbm.at[pl.ds(c * rows_per_chunk, rows_per_chunk)], idx_v)
            copies = [
                pltpu.async_copy(
                    table_hbm.at[idx_v.at[j]],
                    rows_v.at[pl.ds(j * _G, _G)],
                    sem,
                )
                for j in range(rows_per_chunk)
            ]
            for cp in copies:
                cp.wait()
            for q in range(4):
                pltpu.sync_copy(
                    rows_v.at[pl.ds(512 * q, 512)],
                    out_hbm.at[pl.ds(co, 512), pl.ds(32 * q, 32)],
                )
            return carry

        lax.fori_loop(0, chunks_per_w, body, 0)

    return gather(idx2d, table)


def kernel(indices, table):
    b, l = indices.shape
    n = b * l
    v = table.shape[0]
    vp = ((v + _PB - 1) // _PB) * _PB
    table_lin = _prep_table(jnp.transpose(table), v).reshape(vp, _DIM)
    idx = jnp.transpose(indices).astype(jnp.int32)
    jp = (idx & ~2047) + ((idx & 511) << 2) + ((idx & 2047) >> 9)
    idx2d = jp.reshape(n // _G, _G)
    l_pp = l // _P
    acc = None
    for p in range(_P):
        g128_p = _gather_rows(idx2d, table_lin, n // _G, p, _P)
        acc = _transpose_piece(g128_p, acc, p, l_pp, l, b)
    return jnp.transpose(acc, (2, 0, 1))
